# baseline (speedup 1.0000x reference)
import functools

import jax
import jax.numpy as jnp
from jax import lax
from jax.experimental import pallas as pl
from jax.experimental.pallas import tpu as pltpu
from jax.experimental.pallas import tpu_sc as plsc

N = 10000
D = 256
DH = 128
E = 160000
HALF = N // 2
DUMP = HALF
NPADH = 5248
RPTH = NPADH // 16
EP = 163840
EPT = EP // 16
HPE = EPT // 2
RCH = 512
NRC = HPE // RCH
CH2 = 64
CAP = HPE + CH2

NPAD = 10240
CHD = 128
NCHD = EPT // CHD

_mesh = plsc.VectorSubcoreMesh(core_axis_name="c", subcore_axis_name="s")


def _segsum_body(x_a, x_b, gidx_a, gidx_b, sa_lo, sa_hi, sb_lo, sb_hi,
                 outa0, outa1, outb0, outb1,
                 gcomp, scomp, rg, rs, gi_a, si_a, gi_b, si_b,
                 rows_a, rows_b, acc_sh, gsem_a, gsem_b):
    cid = lax.axis_index("c")
    sid = lax.axis_index("s")
    z16 = jnp.zeros((16,), jnp.float32)
    zi16 = jnp.zeros((16,), jnp.int32)
    dump16 = jnp.full((16,), DUMP, jnp.int32)

    def zero_acc():
        def zrow(i, _):
            r = i // 16
            rem = i % 16
            rows_a[r, rem // 8, pl.ds((rem % 8) * 16, 16)] = z16
            return 0
        lax.fori_loop(0, CH2 * 16, zrow, 0)

        def zcp(z, _):
            pltpu.sync_copy(rows_a, acc_sh.at[pl.ds(sid * RPTH + z * CH2, CH2)])
            return 0
        lax.fori_loop(0, RPTH // CH2, zcp, 0)
        pltpu.sync_copy(rows_a.at[pl.ds(0, RPTH % CH2)],
                        acc_sh.at[pl.ds(sid * RPTH + (RPTH // CH2) * CH2,
                                        RPTH % CH2)])

    zero_acc()
    plsc.subcore_barrier()

    def run_direction(x, gidx, sidx_lo, sidx_hi):
        def prep_idx(g, gi, si):
            for q in range(CH2 // 16):
                gi[pl.ds(q * 16, 16)] = gcomp[pl.ds(g * CH2 + q * 16, 16)]
                si[pl.ds(q * 16, 16)] = scomp[pl.ds(g * CH2 + q * 16, 16)]

        def gstart(gi, buf, sem):
            pltpu.async_copy(x.at[gi], buf, sem)

        def gwait(buf, sem):
            pltpu.make_async_copy(x.at[pl.ds(0, CH2)], buf, sem).wait()

        def rstart(h, rc, slot, sem):
            base = sid * EPT + h * HPE + rc * RCH
            pltpu.async_copy(gidx.at[pl.ds(base, RCH)], rg.at[slot], sem)

            @pl.when(cid == 0)
            def _():
                pltpu.async_copy(sidx_lo.at[pl.ds(base, RCH)], rs.at[slot], sem)

            @pl.when(cid == 1)
            def _():
                pltpu.async_copy(sidx_hi.at[pl.ds(base, RCH)], rs.at[slot], sem)

        def rwait(slot, sem):
            pltpu.make_async_copy(gidx.at[pl.ds(0, RCH)], rg.at[slot], sem).wait()
            pltpu.make_async_copy(gidx.at[pl.ds(0, RCH)], rs.at[slot], sem).wait()

        for h in range(2):
            rstart(h, 0, 0, gsem_a)

            def raw_chunk(rc, cnt):
                par = rc % 2

                @pl.when(par == 0)
                def _():
                    rwait(0, gsem_a)

                    @pl.when(rc + 1 < NRC)
                    def _():
                        rstart(h, rc + 1, 1, gsem_b)

                @pl.when(par == 1)
                def _():
                    rwait(1, gsem_b)

                    @pl.when(rc + 1 < NRC)
                    def _():
                        rstart(h, rc + 1, 0, gsem_a)

                def lane(j, c):
                    gv = rg[par, pl.ds(j * 16, 16)]
                    sv = rs[par, pl.ds(j * 16, 16)]
                    m = sv < DUMP
                    plsc.store_compressed(gcomp.at[pl.ds(c, 16)], gv, mask=m)
                    plsc.store_compressed(scomp.at[pl.ds(c, 16)], sv, mask=m)
                    return c + jnp.sum(m.astype(jnp.int32))
                return lax.fori_loop(0, RCH // 16, lane, cnt)
            cnt = lax.fori_loop(0, NRC, raw_chunk, jnp.int32(0))

            npt = ((cnt + CH2 - 1) // CH2) * CH2
            for it in range(CH2 // 16):
                off = cnt + it * 16

                @pl.when(off < npt)
                def _():
                    gcomp[pl.ds(off, 16)] = zi16
                    scomp[pl.ds(off, 16)] = dump16
            nch = npt // CH2

            @pl.when(nch > 0)
            def _():
                prep_idx(0, gi_a, si_a)
                gstart(gi_a, rows_a, gsem_a)

            def chunk(g, _):
                @pl.when(g % 2 == 0)
                def _():
                    @pl.when(g + 1 < nch)
                    def _():
                        prep_idx(g + 1, gi_b, si_b)
                        gstart(gi_b, rows_b, gsem_b)
                    gwait(rows_a, gsem_a)
                    pltpu.sync_copy(rows_a, acc_sh.at[si_a], add=True)

                @pl.when(g % 2 == 1)
                def _():
                    @pl.when(g + 1 < nch)
                    def _():
                        prep_idx(g + 1, gi_a, si_a)
                        gstart(gi_a, rows_a, gsem_a)
                    gwait(rows_b, gsem_b)
                    pltpu.sync_copy(rows_b, acc_sh.at[si_b], add=True)
                return 0
            lax.fori_loop(0, nch, chunk, 0)

    def writeout(out0, out1):
        @pl.when(cid == 0)
        def _():
            pltpu.sync_copy(acc_sh.at[pl.ds(sid * RPTH, RPTH)],
                            out0.at[pl.ds(sid * RPTH, RPTH)])

        @pl.when(cid == 1)
        def _():
            pltpu.sync_copy(acc_sh.at[pl.ds(sid * RPTH, RPTH)],
                            out1.at[pl.ds(sid * RPTH, RPTH)])

    run_direction(x_a, gidx_a, sa_lo, sa_hi)
    plsc.subcore_barrier()
    writeout(outa0, outa1)
    zero_acc()
    plsc.subcore_barrier()
    run_direction(x_b, gidx_b, sb_lo, sb_hi)
    plsc.subcore_barrier()
    writeout(outb0, outb1)


_segsum = pl.kernel(
    _segsum_body,
    out_type=(jax.ShapeDtypeStruct((NPADH, 2, DH), jnp.float32),) * 4,
    mesh=_mesh,
    scratch_types=[
        pltpu.VMEM((CAP,), jnp.int32),
        pltpu.VMEM((CAP,), jnp.int32),
        pltpu.VMEM((2, RCH), jnp.int32),
        pltpu.VMEM((2, RCH), jnp.int32),
        pltpu.VMEM((CH2,), jnp.int32),
        pltpu.VMEM((CH2,), jnp.int32),
        pltpu.VMEM((CH2,), jnp.int32),
        pltpu.VMEM((CH2,), jnp.int32),
        pltpu.VMEM((CH2, 2, DH), jnp.float32),
        pltpu.VMEM((CH2, 2, DH), jnp.float32),
        pltpu.VMEM_SHARED((NPADH, 2, DH), jnp.float32),
        pltpu.SemaphoreType.DMA,
        pltpu.SemaphoreType.DMA,
    ],
    compiler_params=pltpu.CompilerParams(needs_layout_passes=False),
)


def _deg_body(idx_a, idx_b, out_a, out_b, hist_v, idx_v, rowbuf_v, out_v, stage_sh):
    cid = lax.axis_index("c")
    sid = lax.axis_index("s")
    z16 = jnp.zeros((16,), jnp.float32)
    ones16 = jnp.full((16,), 1.0, jnp.float32)

    def zh(i, _):
        hist_v[pl.ds(i * 16, 16)] = z16
        return 0
    lax.fori_loop(0, NPAD // 16, zh, 0)

    def chunk(g, _):
        base = sid * EPT + g * CHD

        @pl.when(cid == 0)
        def _():
            pltpu.sync_copy(idx_a.at[pl.ds(base, CHD)], idx_v)

        @pl.when(cid == 1)
        def _():
            pltpu.sync_copy(idx_b.at[pl.ds(base, CHD)], idx_v)

        def inner(j, _):
            iv = idx_v[pl.ds(j * 16, 16)]
            plsc.addupdate_scatter(hist_v, [iv], ones16)
            return 0
        lax.fori_loop(0, CHD // 16, inner, 0)
        return 0
    lax.fori_loop(0, NCHD, chunk, 0)

    pltpu.sync_copy(hist_v, stage_sh.at[sid])
    plsc.subcore_barrier()

    rpt = NPAD // 16

    def zo(j, _):
        out_v[pl.ds(j * 16, 16)] = z16
        return 0
    lax.fori_loop(0, rpt // 16, zo, 0)
    for r in range(16):
        pltpu.sync_copy(stage_sh.at[r, pl.ds(sid * rpt, rpt)], rowbuf_v)

        def addj(j, _):
            sl = pl.ds(j * 16, 16)
            out_v[sl] = out_v[sl] + rowbuf_v[sl]
            return 0
        lax.fori_loop(0, rpt // 16, addj, 0)

    @pl.when(cid == 0)
    def _():
        pltpu.sync_copy(out_v, out_a.at[pl.ds(sid * rpt, rpt)])

    @pl.when(cid == 1)
    def _():
        pltpu.sync_copy(out_v, out_b.at[pl.ds(sid * rpt, rpt)])


_deg = pl.kernel(
    _deg_body,
    out_type=(jax.ShapeDtypeStruct((NPAD,), jnp.float32),
              jax.ShapeDtypeStruct((NPAD,), jnp.float32)),
    mesh=_mesh,
    scratch_types=[
        pltpu.VMEM((NPAD,), jnp.float32),
        pltpu.VMEM((CHD,), jnp.int32),
        pltpu.VMEM((NPAD // 16,), jnp.float32),
        pltpu.VMEM((NPAD // 16,), jnp.float32),
        pltpu.VMEM_SHARED((16, NPAD), jnp.float32),
    ],
    compiler_params=pltpu.CompilerParams(needs_layout_passes=False),
)

BN = 1000


def _mm_body(relu, agg_lo, agg_hi, degr, x, wl, wr, b, out):
    i = pl.program_id(0)
    agg = jnp.where(i < 5, agg_lo[...], agg_hi[...])
    rdeg = 1.0 / jnp.maximum(degr[...], 1.0)
    mean = agg * rdeg
    dn = (((1,), (1,)), ((), ()))
    acc = lax.dot_general(mean, wl[...], dn, preferred_element_type=jnp.float32)
    acc = acc + lax.dot_general(x[...], wr[...], dn,
                                preferred_element_type=jnp.float32)
    acc = acc + b[...]
    if relu:
        acc = jnp.maximum(acc, 0.0)
    out[...] = acc


def _mm(agg_lo, agg_hi, degr, x, wl, wr, b, relu):
    in_specs = [
        pl.BlockSpec((BN, D), lambda i: (jnp.minimum(i, 4), 0)),
        pl.BlockSpec((BN, D), lambda i: (jnp.maximum(i - 5, 0), 0)),
        pl.BlockSpec((BN, 1), lambda i: (i, 0)),
        pl.BlockSpec((BN, D), lambda i: (i, 0)),
        pl.BlockSpec((D, D), lambda i: (0, 0)),
        pl.BlockSpec((D, D), lambda i: (0, 0)),
        pl.BlockSpec((1, D), lambda i: (0, 0)),
    ]
    return pl.pallas_call(
        functools.partial(_mm_body, relu),
        grid=(N // BN,),
        in_specs=in_specs,
        out_specs=pl.BlockSpec((BN, D), lambda i: (i, 0)),
        out_shape=jax.ShapeDtypeStruct((N, D), jnp.float32),
    )(agg_lo, agg_hi, degr, x, wl, wr, b)


def kernel(s, t, edge_index,
           Wl_s0, bl_s0, Wr_s0, Wl_t0, bl_t0, Wr_t0,
           Wl_s1, bl_s1, Wr_s1, Wl_t1, bl_t1, Wr_t1):
    src = edge_index[0]
    dst = edge_index[1]
    npad_e = EP - E
    pad_g = jnp.zeros((npad_e,), jnp.int32)
    pad_d = jnp.full((npad_e,), DUMP, jnp.int32)
    pad_n = jnp.full((npad_e,), N, jnp.int32)

    src_g = jnp.concatenate([src, pad_g])
    dst_g = jnp.concatenate([dst, pad_g])
    dst_lo = jnp.concatenate([jnp.where(dst < HALF, dst, DUMP), pad_d])
    dst_hi = jnp.concatenate([jnp.where(dst >= HALF, dst - HALF, DUMP), pad_d])
    src_lo = jnp.concatenate([jnp.where(src < HALF, src, DUMP), pad_d])
    src_hi = jnp.concatenate([jnp.where(src >= HALF, src - HALF, DUMP), pad_d])
    dst_f = jnp.concatenate([dst, pad_n])
    src_f = jnp.concatenate([src, pad_n])

    deg_d_p, deg_s_p = _deg(dst_f, src_f)
    deg_d = deg_d_p[:N].reshape(N, 1)
    deg_s = deg_s_p[:N].reshape(N, 1)

    t3 = t.reshape(N, 2, DH)
    s3 = s.reshape(N, 2, DH)
    as0_lo, as0_hi, at0_lo, at0_hi = _segsum(
        t3, s3, src_g, dst_g, dst_lo, dst_hi, src_lo, src_hi)

    s1 = _mm(as0_lo.reshape(NPADH, D), as0_hi.reshape(NPADH, D), deg_d, t, Wl_s0, Wr_s0,
             bl_s0.reshape(1, D), relu=True)
    t1 = _mm(at0_lo.reshape(NPADH, D), at0_hi.reshape(NPADH, D), deg_s, s, Wl_t0, Wr_t0,
             bl_t0.reshape(1, D), relu=True)

    as1_lo, as1_hi, at1_lo, at1_hi = _segsum(
        t1.reshape(N, 2, DH), s1.reshape(N, 2, DH),
        src_g, dst_g, dst_lo, dst_hi, src_lo, src_hi)

    s_out = _mm(as1_lo.reshape(NPADH, D), as1_hi.reshape(NPADH, D), deg_d, t1, Wl_s1, Wr_s1,
                bl_s1.reshape(1, D), relu=False)
    t_out = _mm(at1_lo.reshape(NPADH, D), at1_hi.reshape(NPADH, D), deg_s, s1, Wl_t1, Wr_t1,
                bl_t1.reshape(1, D), relu=False)
    return (s_out, t_out)


# speedup vs baseline: 1.0432x; 1.0432x over previous
import functools

import jax
import jax.numpy as jnp
from jax import lax
from jax.experimental import pallas as pl
from jax.experimental.pallas import tpu as pltpu
from jax.experimental.pallas import tpu_sc as plsc

N = 10000
D = 256
DH = 128
E = 160000
HALF = N // 2
DUMP = HALF
NPADH = 5248
RPTH = NPADH // 16
EP = 163840
EPT = EP // 16
HPE = EPT // 2
RCH = 512
NRC = HPE // RCH
CH2 = 64
CAP = HPE + CH2

NPAD = 10240
CHD = 128
NCHD = EPT // CHD

_mesh = plsc.VectorSubcoreMesh(core_axis_name="c", subcore_axis_name="s")


def _segsum_body(x, gidx, sidx_lo, sidx_hi, out0, out1,
                 gcomp, scomp, rg, rs, gi_a, si_a, gi_b, si_b,
                 rows_a, rows_b, acc_sh, gsem_a, gsem_b):
    cid = lax.axis_index("c")
    sid = lax.axis_index("s")
    z16 = jnp.zeros((16,), jnp.float32)
    zi16 = jnp.zeros((16,), jnp.int32)
    dump16 = jnp.full((16,), DUMP, jnp.int32)

    def zrow(i, _):
        r = i // 16
        rem = i % 16
        rows_a[r, rem // 8, pl.ds((rem % 8) * 16, 16)] = z16
        return 0
    lax.fori_loop(0, CH2 * 16, zrow, 0)

    def zcp(z, _):
        pltpu.sync_copy(rows_a, acc_sh.at[pl.ds(sid * RPTH + z * CH2, CH2)])
        return 0
    lax.fori_loop(0, RPTH // CH2, zcp, 0)
    pltpu.sync_copy(rows_a.at[pl.ds(0, RPTH % CH2)],
                    acc_sh.at[pl.ds(sid * RPTH + (RPTH // CH2) * CH2,
                                    RPTH % CH2)])
    plsc.subcore_barrier()

    def prep_idx(g, gi, si):
        for q in range(CH2 // 16):
            gi[pl.ds(q * 16, 16)] = gcomp[pl.ds(g * CH2 + q * 16, 16)]
            si[pl.ds(q * 16, 16)] = scomp[pl.ds(g * CH2 + q * 16, 16)]

    def gstart(gi, buf, sem):
        pltpu.async_copy(x.at[gi], buf, sem)

    def gwait(buf, sem):
        pltpu.make_async_copy(x.at[pl.ds(0, CH2)], buf, sem).wait()

    def rstart(h, rc, slot, sem):
        base = sid * EPT + h * HPE + rc * RCH
        pltpu.async_copy(gidx.at[pl.ds(base, RCH)], rg.at[slot], sem)

        @pl.when(cid == 0)
        def _():
            pltpu.async_copy(sidx_lo.at[pl.ds(base, RCH)], rs.at[slot], sem)

        @pl.when(cid == 1)
        def _():
            pltpu.async_copy(sidx_hi.at[pl.ds(base, RCH)], rs.at[slot], sem)

    def rwait(slot, sem):
        pltpu.make_async_copy(gidx.at[pl.ds(0, RCH)], rg.at[slot], sem).wait()
        pltpu.make_async_copy(gidx.at[pl.ds(0, RCH)], rs.at[slot], sem).wait()

    for h in range(2):
        rstart(h, 0, 0, gsem_a)

        def raw_chunk(rc, cnt):
            par = rc % 2

            @pl.when(par == 0)
            def _():
                rwait(0, gsem_a)

                @pl.when(rc + 1 < NRC)
                def _():
                    rstart(h, rc + 1, 1, gsem_b)

            @pl.when(par == 1)
            def _():
                rwait(1, gsem_b)

                @pl.when(rc + 1 < NRC)
                def _():
                    rstart(h, rc + 1, 0, gsem_a)

            def lane(j, c):
                gv = rg[par, pl.ds(j * 16, 16)]
                sv = rs[par, pl.ds(j * 16, 16)]
                m = sv < DUMP
                plsc.store_compressed(gcomp.at[pl.ds(c, 16)], gv, mask=m)
                plsc.store_compressed(scomp.at[pl.ds(c, 16)], sv, mask=m)
                return c + jnp.sum(m.astype(jnp.int32))
            return lax.fori_loop(0, RCH // 16, lane, cnt)
        cnt = lax.fori_loop(0, NRC, raw_chunk, jnp.int32(0))

        npt = ((cnt + CH2 - 1) // CH2) * CH2
        for it in range(CH2 // 16):
            off = cnt + it * 16

            @pl.when(off < npt)
            def _():
                gcomp[pl.ds(off, 16)] = zi16
                scomp[pl.ds(off, 16)] = dump16
        nch = npt // CH2

        @pl.when(nch > 0)
        def _():
            prep_idx(0, gi_a, si_a)
            gstart(gi_a, rows_a, gsem_a)

        def chunk(g, _):
            @pl.when(g % 2 == 0)
            def _():
                @pl.when(g + 1 < nch)
                def _():
                    prep_idx(g + 1, gi_b, si_b)
                    gstart(gi_b, rows_b, gsem_b)
                gwait(rows_a, gsem_a)
                pltpu.sync_copy(rows_a, acc_sh.at[si_a], add=True)

            @pl.when(g % 2 == 1)
            def _():
                @pl.when(g + 1 < nch)
                def _():
                    prep_idx(g + 1, gi_a, si_a)
                    gstart(gi_a, rows_a, gsem_a)
                gwait(rows_b, gsem_b)
                pltpu.sync_copy(rows_b, acc_sh.at[si_b], add=True)
            return 0
        lax.fori_loop(0, nch, chunk, 0)

    plsc.subcore_barrier()

    @pl.when(cid == 0)
    def _():
        pltpu.sync_copy(acc_sh.at[pl.ds(sid * RPTH, RPTH)],
                        out0.at[pl.ds(sid * RPTH, RPTH)])

    @pl.when(cid == 1)
    def _():
        pltpu.sync_copy(acc_sh.at[pl.ds(sid * RPTH, RPTH)],
                        out1.at[pl.ds(sid * RPTH, RPTH)])


_segsum = pl.kernel(
    _segsum_body,
    out_type=(jax.ShapeDtypeStruct((NPADH, 2, DH), jnp.float32),
              jax.ShapeDtypeStruct((NPADH, 2, DH), jnp.float32)),
    mesh=_mesh,
    scratch_types=[
        pltpu.VMEM((CAP,), jnp.int32),
        pltpu.VMEM((CAP,), jnp.int32),
        pltpu.VMEM((2, RCH), jnp.int32),
        pltpu.VMEM((2, RCH), jnp.int32),
        pltpu.VMEM((CH2,), jnp.int32),
        pltpu.VMEM((CH2,), jnp.int32),
        pltpu.VMEM((CH2,), jnp.int32),
        pltpu.VMEM((CH2,), jnp.int32),
        pltpu.VMEM((CH2, 2, DH), jnp.float32),
        pltpu.VMEM((CH2, 2, DH), jnp.float32),
        pltpu.VMEM_SHARED((NPADH, 2, DH), jnp.float32),
        pltpu.SemaphoreType.DMA,
        pltpu.SemaphoreType.DMA,
    ],
    compiler_params=pltpu.CompilerParams(needs_layout_passes=False),
)


def _deg_body(idx_a, idx_b, out_a, out_b, hist_v, idx_v, rowbuf_v, out_v, stage_sh):
    cid = lax.axis_index("c")
    sid = lax.axis_index("s")
    z16 = jnp.zeros((16,), jnp.float32)
    ones16 = jnp.full((16,), 1.0, jnp.float32)

    def zh(i, _):
        hist_v[pl.ds(i * 16, 16)] = z16
        return 0
    lax.fori_loop(0, NPAD // 16, zh, 0)

    def chunk(g, _):
        base = sid * EPT + g * CHD

        @pl.when(cid == 0)
        def _():
            pltpu.sync_copy(idx_a.at[pl.ds(base, CHD)], idx_v)

        @pl.when(cid == 1)
        def _():
            pltpu.sync_copy(idx_b.at[pl.ds(base, CHD)], idx_v)

        def inner(j, _):
            iv = idx_v[pl.ds(j * 16, 16)]
            plsc.addupdate_scatter(hist_v, [iv], ones16)
            return 0
        lax.fori_loop(0, CHD // 16, inner, 0)
        return 0
    lax.fori_loop(0, NCHD, chunk, 0)

    pltpu.sync_copy(hist_v, stage_sh.at[sid])
    plsc.subcore_barrier()

    rpt = NPAD // 16

    def zo(j, _):
        out_v[pl.ds(j * 16, 16)] = z16
        return 0
    lax.fori_loop(0, rpt // 16, zo, 0)
    for r in range(16):
        pltpu.sync_copy(stage_sh.at[r, pl.ds(sid * rpt, rpt)], rowbuf_v)

        def addj(j, _):
            sl = pl.ds(j * 16, 16)
            out_v[sl] = out_v[sl] + rowbuf_v[sl]
            return 0
        lax.fori_loop(0, rpt // 16, addj, 0)

    @pl.when(cid == 0)
    def _():
        pltpu.sync_copy(out_v, out_a.at[pl.ds(sid * rpt, rpt)])

    @pl.when(cid == 1)
    def _():
        pltpu.sync_copy(out_v, out_b.at[pl.ds(sid * rpt, rpt)])


_deg = pl.kernel(
    _deg_body,
    out_type=(jax.ShapeDtypeStruct((NPAD,), jnp.float32),
              jax.ShapeDtypeStruct((NPAD,), jnp.float32)),
    mesh=_mesh,
    scratch_types=[
        pltpu.VMEM((NPAD,), jnp.float32),
        pltpu.VMEM((CHD,), jnp.int32),
        pltpu.VMEM((NPAD // 16,), jnp.float32),
        pltpu.VMEM((NPAD // 16,), jnp.float32),
        pltpu.VMEM_SHARED((16, NPAD), jnp.float32),
    ],
    compiler_params=pltpu.CompilerParams(needs_layout_passes=False),
)

BN = 1000


def _mm_body(relu, agg_lo, agg_hi, degr, x, wl, wr, b, out):
    i = pl.program_id(0)
    agg = jnp.where(i < 5, agg_lo[...], agg_hi[...])
    rdeg = 1.0 / jnp.maximum(degr[...], 1.0)
    mean = agg * rdeg
    dn = (((1,), (1,)), ((), ()))
    acc = lax.dot_general(mean, wl[...], dn, preferred_element_type=jnp.float32)
    acc = acc + lax.dot_general(x[...], wr[...], dn,
                                preferred_element_type=jnp.float32)
    acc = acc + b[...]
    if relu:
        acc = jnp.maximum(acc, 0.0)
    out[...] = acc


def _mm(agg_lo, agg_hi, degr, x, wl, wr, b, relu):
    in_specs = [
        pl.BlockSpec((BN, D), lambda i: (jnp.minimum(i, 4), 0)),
        pl.BlockSpec((BN, D), lambda i: (jnp.maximum(i - 5, 0), 0)),
        pl.BlockSpec((BN, 1), lambda i: (i, 0)),
        pl.BlockSpec((BN, D), lambda i: (i, 0)),
        pl.BlockSpec((D, D), lambda i: (0, 0)),
        pl.BlockSpec((D, D), lambda i: (0, 0)),
        pl.BlockSpec((1, D), lambda i: (0, 0)),
    ]
    return pl.pallas_call(
        functools.partial(_mm_body, relu),
        grid=(N // BN,),
        in_specs=in_specs,
        out_specs=pl.BlockSpec((BN, D), lambda i: (i, 0)),
        out_shape=jax.ShapeDtypeStruct((N, D), jnp.float32),
    )(agg_lo, agg_hi, degr, x, wl, wr, b)


def kernel(s, t, edge_index,
           Wl_s0, bl_s0, Wr_s0, Wl_t0, bl_t0, Wr_t0,
           Wl_s1, bl_s1, Wr_s1, Wl_t1, bl_t1, Wr_t1):
    src = edge_index[0]
    dst = edge_index[1]
    npad_e = EP - E
    pad_g = jnp.zeros((npad_e,), jnp.int32)
    pad_d = jnp.full((npad_e,), DUMP, jnp.int32)
    pad_n = jnp.full((npad_e,), N, jnp.int32)

    src_g = jnp.concatenate([src, pad_g])
    dst_g = jnp.concatenate([dst, pad_g])
    dst_lo = jnp.concatenate([jnp.where(dst < HALF, dst, DUMP), pad_d])
    dst_hi = jnp.concatenate([jnp.where(dst >= HALF, dst - HALF, DUMP), pad_d])
    src_lo = jnp.concatenate([jnp.where(src < HALF, src, DUMP), pad_d])
    src_hi = jnp.concatenate([jnp.where(src >= HALF, src - HALF, DUMP), pad_d])
    dst_f = jnp.concatenate([dst, pad_n])
    src_f = jnp.concatenate([src, pad_n])

    deg_d_p, deg_s_p = _deg(dst_f, src_f)
    deg_d = deg_d_p[:N].reshape(N, 1)
    deg_s = deg_s_p[:N].reshape(N, 1)

    t3 = t.reshape(N, 2, DH)
    s3 = s.reshape(N, 2, DH)
    as0_lo, as0_hi = _segsum(t3, src_g, dst_lo, dst_hi)
    at0_lo, at0_hi = _segsum(s3, dst_g, src_lo, src_hi)

    s1 = _mm(as0_lo.reshape(NPADH, D), as0_hi.reshape(NPADH, D), deg_d, t, Wl_s0, Wr_s0,
             bl_s0.reshape(1, D), relu=True)
    t1 = _mm(at0_lo.reshape(NPADH, D), at0_hi.reshape(NPADH, D), deg_s, s, Wl_t0, Wr_t0,
             bl_t0.reshape(1, D), relu=True)

    as1_lo, as1_hi = _segsum(t1.reshape(N, 2, DH), src_g, dst_lo, dst_hi)
    at1_lo, at1_hi = _segsum(s1.reshape(N, 2, DH), dst_g, src_lo, src_hi)

    s_out = _mm(as1_lo.reshape(NPADH, D), as1_hi.reshape(NPADH, D), deg_d, t1, Wl_s1, Wr_s1,
                bl_s1.reshape(1, D), relu=False)
    t_out = _mm(at1_lo.reshape(NPADH, D), at1_hi.reshape(NPADH, D), deg_s, s1, Wl_t1, Wr_t1,
                bl_t1.reshape(1, D), relu=False)
    return (s_out, t_out)
